# Initial kernel scaffold; baseline (speedup 1.0000x reference)
import jax
import jax.numpy as jnp
from jax.experimental import pallas as pl


def kernel(x, edge_index, W_neigh, b_neigh, W_root, b_root):
    raise NotImplementedError("write your pallas kernel here")


# speedup vs baseline: 9.1988x; 9.1988x over previous
import functools

import jax
import jax.numpy as jnp
from jax import lax
from jax.experimental import pallas as pl
from jax.experimental.pallas import tpu as pltpu
from jax.experimental.pallas import tpu_sc as plsc

N = 10000
DIN = 128
E = 320000
NC = 2
NS = 16
L = 16
DH = DIN // NC
DHA = DH + L
K = 128
R = 10240
RPT = R // NS
NCHUNK = 158
EPT = NCHUNK * K
EPAD = NS * EPT


def _sc_aggregate(x_halves, src2d, dst2d, zeros_tile):
    mesh = plsc.VectorSubcoreMesh(core_axis_name="c", subcore_axis_name="s")

    @functools.partial(
        pl.kernel,
        mesh=mesh,
        compiler_params=pltpu.CompilerParams(use_tc_tiling_on_sc=False),
        out_type=jax.ShapeDtypeStruct((NC, R, DHA), jnp.float32),
        scratch_types=[
            pltpu.VMEM((NCHUNK, K), jnp.int32),
            pltpu.VMEM((NCHUNK, K), jnp.int32),
            pltpu.VMEM((K, DHA), jnp.float32),
            pltpu.VMEM((K, DHA), jnp.float32),
            pltpu.VMEM_SHARED((R, DHA), jnp.float32),
            pltpu.SemaphoreType.DMA,
            pltpu.SemaphoreType.DMA,
        ],
    )
    def body(x_hbm, src_hbm, dst_hbm, z_hbm, out_hbm,
             sidx, didx, rows0, rows1, agg, sem0, sem1):
        c = lax.axis_index("c")
        s = lax.axis_index("s")
        xh = x_hbm.at[c]
        pltpu.sync_copy(src_hbm.at[pl.ds(s * NCHUNK, NCHUNK)], sidx)
        pltpu.sync_copy(dst_hbm.at[pl.ds(s * NCHUNK, NCHUNK)], didx)
        pltpu.sync_copy(z_hbm, agg.at[pl.ds(s * RPT, RPT)])
        plsc.subcore_barrier()

        pltpu.async_copy(xh.at[sidx.at[0]], rows0, sem0)
        pltpu.async_copy(xh.at[sidx.at[1]], rows1, sem1)

        @pl.loop(0, NCHUNK, step=2)
        def _(j):
            pltpu.make_async_copy(xh.at[sidx.at[0]], rows0, sem0).wait()
            pltpu.sync_copy(rows0, agg.at[didx.at[j]], add=True)

            @pl.when(j + 2 < NCHUNK)
            def _():
                pltpu.async_copy(xh.at[sidx.at[j + 2]], rows0, sem0)

            pltpu.make_async_copy(xh.at[sidx.at[1]], rows1, sem1).wait()
            pltpu.sync_copy(rows1, agg.at[didx.at[j + 1]], add=True)

            @pl.when(j + 3 < NCHUNK)
            def _():
                pltpu.async_copy(xh.at[sidx.at[j + 3]], rows1, sem1)

        plsc.subcore_barrier()
        pltpu.sync_copy(agg.at[pl.ds(s * RPT, RPT)],
                        out_hbm.at[c, pl.ds(s * RPT, RPT)])

    return body(x_halves, src2d, dst2d, zeros_tile)


def _tc_combine(agg, x, wnT, wrT, bias):
    BR = 400
    nblk = N // BR

    def body(a_ref, x_ref, wn_ref, wr_ref, b_ref, o_ref):
        deg = jnp.maximum(a_ref[0, :, DH:DH + 1], 1.0)
        ssum = jnp.concatenate([a_ref[0, :, :DH], a_ref[1, :, :DH]], axis=1)
        neigh = ssum / deg
        o_ref[...] = (
            jnp.dot(neigh, wn_ref[...], precision=lax.Precision.HIGHEST,
                    preferred_element_type=jnp.float32)
            + jnp.dot(x_ref[...], wr_ref[...], precision=lax.Precision.HIGHEST,
                      preferred_element_type=jnp.float32)
            + b_ref[...]
        )

    return pl.pallas_call(
        body,
        grid=(nblk,),
        in_specs=[
            pl.BlockSpec((NC, BR, DHA), lambda i: (0, i, 0)),
            pl.BlockSpec((BR, DIN), lambda i: (i, 0)),
            pl.BlockSpec((DIN, DIN), lambda i: (0, 0)),
            pl.BlockSpec((DIN, DIN), lambda i: (0, 0)),
            pl.BlockSpec((1, DIN), lambda i: (0, 0)),
        ],
        out_specs=pl.BlockSpec((BR, DIN), lambda i: (i, 0)),
        out_shape=jax.ShapeDtypeStruct((N, DIN), jnp.float32),
    )(agg, x, wnT, wrT, bias)


def kernel(x, edge_index, W_neigh, b_neigh, W_root, b_root):
    f32 = jnp.float32
    src = edge_index[0].astype(jnp.int32)
    dst = edge_index[1].astype(jnp.int32)
    npad = EPAD - E
    ar = jnp.arange(npad, dtype=jnp.int32)
    src = jnp.concatenate([src, ar % N])
    dst = jnp.concatenate([dst, N + ar % (R - N)])
    src2d = src.reshape(NS * NCHUNK, K)
    dst2d = dst.reshape(NS * NCHUNK, K)
    x = x.astype(f32)
    ones = jnp.ones((N, 1), f32)
    zpad = jnp.zeros((N, DHA - DH - 1), f32)
    x_halves = jnp.stack([
        jnp.concatenate([x[:, :DH], ones, zpad], axis=1),
        jnp.concatenate([x[:, DH:], ones, zpad], axis=1),
    ])
    zeros_tile = jnp.zeros((RPT, DHA), f32)
    agg = _sc_aggregate(x_halves, src2d, dst2d, zeros_tile)
    bias = (b_neigh + b_root).reshape(1, DIN).astype(f32)
    return _tc_combine(agg, x, W_neigh.T.astype(f32), W_root.T.astype(f32), bias)


# speedup vs baseline: 9.5300x; 1.0360x over previous
import functools

import jax
import jax.numpy as jnp
from jax import lax
from jax.experimental import pallas as pl
from jax.experimental.pallas import tpu as pltpu
from jax.experimental.pallas import tpu_sc as plsc

N = 10000
DIN = 128
E = 320000
NC = 2
NS = 16
L = 16
DH = DIN // NC
DHA = DH + L
K = 125
NBUF = 4
R = N
RPT = R // NS
NCHUNK = 160
EPT = NCHUNK * K


def _sc_aggregate(x_halves, src2d, dst2d, zeros_tile):
    mesh = plsc.VectorSubcoreMesh(core_axis_name="c", subcore_axis_name="s")

    @functools.partial(
        pl.kernel,
        mesh=mesh,
        compiler_params=pltpu.CompilerParams(use_tc_tiling_on_sc=False),
        out_type=jax.ShapeDtypeStruct((NC, R, DHA), jnp.float32),
        scratch_types=[
            pltpu.VMEM((NCHUNK, K), jnp.int32),
            pltpu.VMEM((NCHUNK, K), jnp.int32),
            *[pltpu.VMEM((K, DHA), jnp.float32) for _ in range(NBUF)],
            pltpu.VMEM_SHARED((R, DHA), jnp.float32),
            *[pltpu.SemaphoreType.DMA for _ in range(2 * NBUF)],
        ],
    )
    def body(x_hbm, src_hbm, dst_hbm, z_hbm, out_hbm, sidx, didx, *bufs):
        rows = bufs[:NBUF]
        agg = bufs[NBUF]
        gsem = bufs[NBUF + 1:NBUF + 1 + NBUF]
        ssem = bufs[NBUF + 1 + NBUF:]
        c = lax.axis_index("c")
        s = lax.axis_index("s")
        xh = x_hbm.at[c]
        pltpu.sync_copy(src_hbm.at[pl.ds(s * NCHUNK, NCHUNK)], sidx)
        pltpu.sync_copy(dst_hbm.at[pl.ds(s * NCHUNK, NCHUNK)], didx)
        pltpu.sync_copy(z_hbm, agg.at[pl.ds(s * RPT, RPT)])
        plsc.subcore_barrier()

        for b in range(NBUF):
            pltpu.async_copy(xh.at[sidx.at[b]], rows[b], gsem[b])

        @pl.loop(0, NCHUNK, step=NBUF)
        def _(j):
            for b in range(NBUF):
                pltpu.make_async_copy(xh.at[sidx.at[0]], rows[b], gsem[b]).wait()
                pltpu.async_copy(rows[b], agg.at[didx.at[j + b]], ssem[b],
                                 add=True)
            for b in range(NBUF):
                pltpu.make_async_copy(rows[b], agg.at[didx.at[0]],
                                      ssem[b]).wait()

                @pl.when(j + NBUF + b < NCHUNK)
                def _():
                    pltpu.async_copy(xh.at[sidx.at[j + NBUF + b]], rows[b],
                                     gsem[b])

        plsc.subcore_barrier()
        pltpu.sync_copy(agg.at[pl.ds(s * RPT, RPT)],
                        out_hbm.at[c, pl.ds(s * RPT, RPT)])

    return body(x_halves, src2d, dst2d, zeros_tile)


def _tc_combine(agg, x, wnT, wrT, bias):
    BR = 400
    nblk = N // BR

    def body(a_ref, x_ref, wn_ref, wr_ref, b_ref, o_ref):
        deg = jnp.maximum(a_ref[0, :, DH:DH + 1], 1.0)
        ssum = jnp.concatenate([a_ref[0, :, :DH], a_ref[1, :, :DH]], axis=1)
        neigh = ssum / deg
        o_ref[...] = (
            jnp.dot(neigh, wn_ref[...], preferred_element_type=jnp.float32)
            + jnp.dot(x_ref[...], wr_ref[...], preferred_element_type=jnp.float32)
            + b_ref[...]
        )

    return pl.pallas_call(
        body,
        grid=(nblk,),
        in_specs=[
            pl.BlockSpec((NC, BR, DHA), lambda i: (0, i, 0)),
            pl.BlockSpec((BR, DIN), lambda i: (i, 0)),
            pl.BlockSpec((DIN, DIN), lambda i: (0, 0)),
            pl.BlockSpec((DIN, DIN), lambda i: (0, 0)),
            pl.BlockSpec((1, DIN), lambda i: (0, 0)),
        ],
        out_specs=pl.BlockSpec((BR, DIN), lambda i: (i, 0)),
        out_shape=jax.ShapeDtypeStruct((N, DIN), jnp.float32),
    )(agg, x, wnT, wrT, bias)


def kernel(x, edge_index, W_neigh, b_neigh, W_root, b_root):
    f32 = jnp.float32
    src2d = edge_index[0].astype(jnp.int32).reshape(NS * NCHUNK, K)
    dst2d = edge_index[1].astype(jnp.int32).reshape(NS * NCHUNK, K)
    x = x.astype(f32)
    ones = jnp.ones((N, 1), f32)
    zpad = jnp.zeros((N, DHA - DH - 1), f32)
    x_halves = jnp.stack([
        jnp.concatenate([x[:, :DH], ones, zpad], axis=1),
        jnp.concatenate([x[:, DH:], ones, zpad], axis=1),
    ])
    zeros_tile = jnp.zeros((RPT, DHA), f32)
    agg = _sc_aggregate(x_halves, src2d, dst2d, zeros_tile)
    bias = (b_neigh + b_root).reshape(1, DIN).astype(f32)
    return _tc_combine(agg, x, W_neigh.T.astype(f32), W_root.T.astype(f32), bias)


# speedup vs baseline: 11.5633x; 1.2134x over previous
import dataclasses
import functools

import jax
import jax.numpy as jnp
from jax import lax
from jax.experimental import pallas as pl
from jax.experimental.pallas import tpu as pltpu
from jax.experimental.pallas import tpu_sc as plsc

N = 10000
DIN = 128
E = 320000
NC = 2
NS = 16
L = 16
DH = DIN // NC
K = 125
NBUF = 4
R = N
RPT = R // NS
NCHUNK = 160
EPT = NCHUNK * K
HR = 80
NFULL = K // L
TAIL = K - NFULL * L
TOFF = K - L

_CP = pltpu.CompilerParams(use_tc_tiling_on_sc=False)
if "needs_layout_passes" in pltpu.CompilerParams.__dataclass_fields__:
    _CP = dataclasses.replace(_CP, needs_layout_passes=False)


def _sc_aggregate(x_halves, edge2d, zeros_tile, iota_hr):
    mesh = plsc.VectorSubcoreMesh(core_axis_name="c", subcore_axis_name="s")

    @functools.partial(
        pl.kernel,
        mesh=mesh,
        compiler_params=_CP,
        out_type=[jax.ShapeDtypeStruct((NC, R, DH), jnp.float32),
                  jax.ShapeDtypeStruct((NC, HR, 128), jnp.float32)],
        scratch_types=[
            pltpu.VMEM((NCHUNK, K), jnp.int32),
            pltpu.VMEM((NCHUNK, K), jnp.int32),
            *[pltpu.VMEM((K, DH), jnp.float32) for _ in range(NBUF)],
            pltpu.VMEM((HR, 128), jnp.float32),
            pltpu.VMEM((HR,), jnp.int32),
            pltpu.VMEM_SHARED((R, DH), jnp.float32),
            pltpu.VMEM_SHARED((HR, 128), jnp.float32),
            *[pltpu.SemaphoreType.DMA for _ in range(2 * NBUF)],
        ],
    )
    def body(x_hbm, e_hbm, z_hbm, i_hbm, outa_hbm, outd_hbm,
             sidx, didx, *bufs):
        rows = bufs[:NBUF]
        hist = bufs[NBUF]
        iota_v = bufs[NBUF + 1]
        agg = bufs[NBUF + 2]
        sdeg = bufs[NBUF + 3]
        gsem = bufs[NBUF + 4:NBUF + 4 + NBUF]
        ssem = bufs[NBUF + 4 + NBUF:]
        c = lax.axis_index("c")
        s = lax.axis_index("s")
        xh = x_hbm.at[c]
        ones16 = jnp.ones((L,), jnp.float32)
        zeros16 = jnp.zeros((L,), jnp.float32)
        tailmask = lax.broadcasted_iota(jnp.int32, (L,), 0) >= (L - TAIL)

        pltpu.sync_copy(e_hbm.at[0, pl.ds(s * NCHUNK, NCHUNK)], sidx)
        pltpu.sync_copy(e_hbm.at[1, pl.ds(s * NCHUNK, NCHUNK)], didx)
        pltpu.sync_copy(z_hbm, agg.at[pl.ds(s * RPT, RPT)])
        pltpu.sync_copy(i_hbm, iota_v)

        @pl.loop(0, HR)
        def _(r):
            for v in range(8):
                hist[r, pl.ds(v * L, L)] = zeros16

        @pl.when(s == 0)
        def _():
            pltpu.sync_copy(hist, sdeg)
        plsc.subcore_barrier()

        def hist_row(r):
            for v in range(NFULL):
                idx = didx[r, pl.ds(v * L, L)]
                plsc.addupdate_scatter(hist, [idx >> 7, idx & 127], ones16)
            idx = didx[r, pl.ds(TOFF, L)]
            plsc.addupdate_scatter(hist, [idx >> 7, idx & 127], ones16,
                                   mask=tailmask)

        for b in range(NBUF):
            pltpu.async_copy(xh.at[sidx.at[b]], rows[b], gsem[b])

        @pl.loop(0, NCHUNK, step=NBUF)
        def _(j):
            for b in range(NBUF):
                pltpu.make_async_copy(xh.at[sidx.at[0]], rows[b], gsem[b]).wait()
                pltpu.async_copy(rows[b], agg.at[didx.at[j + b]], ssem[b],
                                 add=True)
            for b in range(NBUF):
                hist_row(j + b)
            for b in range(NBUF):
                pltpu.make_async_copy(rows[b], agg.at[didx.at[0]],
                                      ssem[b]).wait()

                @pl.when(j + NBUF + b < NCHUNK)
                def _():
                    pltpu.async_copy(xh.at[sidx.at[j + NBUF + b]], rows[b],
                                     gsem[b])

        plsc.subcore_barrier()
        pltpu.sync_copy(hist, sdeg.at[iota_v], add=True)
        plsc.subcore_barrier()
        pltpu.sync_copy(agg.at[pl.ds(s * RPT, RPT)],
                        outa_hbm.at[c, pl.ds(s * RPT, RPT)])
        pltpu.sync_copy(sdeg.at[pl.ds(s * (HR // NS), HR // NS)],
                        outd_hbm.at[c, pl.ds(s * (HR // NS), HR // NS)])

    return body(x_halves, edge2d, zeros_tile, iota_hr)


def _tc_combine(agg, deg, x, wnT, wrT, bias):
    BR = 1024
    nblk = (N + BR - 1) // BR

    def body(a_ref, d_ref, x_ref, wn_ref, wr_ref, b_ref, o_ref):
        d = jnp.maximum(d_ref[...], 1.0)
        neigh = jnp.concatenate([a_ref[0], a_ref[1]], axis=1) / d
        o_ref[...] = (
            jnp.dot(neigh, wn_ref[...], preferred_element_type=jnp.float32)
            + jnp.dot(x_ref[...], wr_ref[...], preferred_element_type=jnp.float32)
            + b_ref[...]
        )

    return pl.pallas_call(
        body,
        grid=(nblk,),
        in_specs=[
            pl.BlockSpec((NC, BR, DH), lambda i: (0, i, 0)),
            pl.BlockSpec((BR, 1), lambda i: (i, 0)),
            pl.BlockSpec((BR, DIN), lambda i: (i, 0)),
            pl.BlockSpec((DIN, DIN), lambda i: (0, 0)),
            pl.BlockSpec((DIN, DIN), lambda i: (0, 0)),
            pl.BlockSpec((1, DIN), lambda i: (0, 0)),
        ],
        out_specs=pl.BlockSpec((BR, DIN), lambda i: (i, 0)),
        out_shape=jax.ShapeDtypeStruct((N, DIN), jnp.float32),
    )(agg, deg, x, wnT, wrT, bias)


def kernel(x, edge_index, W_neigh, b_neigh, W_root, b_root):
    f32 = jnp.float32
    edge2d = edge_index.astype(jnp.int32).reshape(2, NS * NCHUNK, K)
    x = x.astype(f32)
    x_halves = x.reshape(N, NC, DH).transpose(1, 0, 2)
    zeros_tile = jnp.zeros((RPT, DH), f32)
    iota_hr = jnp.arange(HR, dtype=jnp.int32)
    agg, deg = _sc_aggregate(x_halves, edge2d, zeros_tile, iota_hr)
    deg = deg[0].reshape(HR * 128, 1)[:N]
    bias = (b_neigh + b_root).reshape(1, DIN).astype(f32)
    return _tc_combine(agg, deg, x, W_neigh.T.astype(f32), W_root.T.astype(f32),
                       bias)


# speedup vs baseline: 11.7562x; 1.0167x over previous
import dataclasses
import functools

import jax
import jax.numpy as jnp
from jax import lax
from jax.experimental import pallas as pl
from jax.experimental.pallas import tpu as pltpu
from jax.experimental.pallas import tpu_sc as plsc

N = 10000
DIN = 128
E = 320000
NC = 2
NS = 16
L = 16
DH = DIN // NC
K = 80
NBUF = 5
R = N
RPT = R // NS
NCHUNK = 250
EPT = NCHUNK * K
HR = 80
NFULL = K // L

_CP = pltpu.CompilerParams(use_tc_tiling_on_sc=False)
if "needs_layout_passes" in pltpu.CompilerParams.__dataclass_fields__:
    _CP = dataclasses.replace(_CP, needs_layout_passes=False)


def _sc_aggregate(x_halves, edges, zeros_tile, iota_hr):
    mesh = plsc.VectorSubcoreMesh(core_axis_name="c", subcore_axis_name="s")

    @functools.partial(
        pl.kernel,
        mesh=mesh,
        compiler_params=_CP,
        out_type=[jax.ShapeDtypeStruct((NC, R, DH), jnp.float32),
                  jax.ShapeDtypeStruct((NC, HR, 128), jnp.float32)],
        scratch_types=[
            pltpu.VMEM((EPT,), jnp.int32),
            pltpu.VMEM((EPT,), jnp.int32),
            *[pltpu.VMEM((K, DH), jnp.float32) for _ in range(NBUF)],
            pltpu.VMEM((HR, 128), jnp.float32),
            pltpu.VMEM((HR,), jnp.int32),
            pltpu.VMEM_SHARED((R, DH), jnp.float32),
            pltpu.VMEM_SHARED((HR, 128), jnp.float32),
            *[pltpu.SemaphoreType.DMA for _ in range(2 * NBUF)],
        ],
    )
    def body(x_hbm, e_hbm, z_hbm, i_hbm, outa_hbm, outd_hbm,
             sidx, didx, *bufs):
        rows = bufs[:NBUF]
        hist = bufs[NBUF]
        iota_v = bufs[NBUF + 1]
        agg = bufs[NBUF + 2]
        sdeg = bufs[NBUF + 3]
        gsem = bufs[NBUF + 4:NBUF + 4 + NBUF]
        ssem = bufs[NBUF + 4 + NBUF:]
        c = lax.axis_index("c")
        s = lax.axis_index("s")
        xh = x_hbm.at[c]
        ones16 = jnp.ones((L,), jnp.float32)
        zeros16 = jnp.zeros((L,), jnp.float32)

        pltpu.sync_copy(e_hbm.at[0, pl.ds(s * EPT, EPT)], sidx)
        pltpu.sync_copy(e_hbm.at[1, pl.ds(s * EPT, EPT)], didx)
        pltpu.sync_copy(z_hbm, agg.at[pl.ds(s * RPT, RPT)])
        pltpu.sync_copy(i_hbm, iota_v)

        @pl.loop(0, HR)
        def _(r):
            for v in range(8):
                hist[r, pl.ds(v * L, L)] = zeros16

        @pl.when(s == 0)
        def _():
            pltpu.sync_copy(hist, sdeg)
        plsc.subcore_barrier()

        def hist_row(r):
            for v in range(NFULL):
                idx = didx[pl.ds(r * K + v * L, L)]
                plsc.addupdate_scatter(hist, [idx >> 7, idx & 127], ones16)

        for b in range(NBUF):
            pltpu.async_copy(xh.at[sidx.at[pl.ds(b * K, K)]], rows[b], gsem[b])

        @pl.loop(0, NCHUNK, step=NBUF)
        def _(j):
            for b in range(NBUF):
                pltpu.make_async_copy(xh.at[sidx.at[pl.ds(0, K)]], rows[b],
                                      gsem[b]).wait()
                pltpu.async_copy(rows[b], agg.at[didx.at[pl.ds((j + b) * K, K)]],
                                 ssem[b], add=True)
            for b in range(NBUF):
                hist_row(j + b)
            for b in range(NBUF):
                pltpu.make_async_copy(rows[b], agg.at[didx.at[pl.ds(0, K)]],
                                      ssem[b]).wait()

                @pl.when(j + NBUF + b < NCHUNK)
                def _():
                    pltpu.async_copy(
                        xh.at[sidx.at[pl.ds((j + NBUF + b) * K, K)]], rows[b],
                        gsem[b])

        plsc.subcore_barrier()
        pltpu.sync_copy(hist, sdeg.at[iota_v], add=True)
        plsc.subcore_barrier()
        pltpu.sync_copy(agg.at[pl.ds(s * RPT, RPT)],
                        outa_hbm.at[c, pl.ds(s * RPT, RPT)])
        pltpu.sync_copy(sdeg.at[pl.ds(s * (HR // NS), HR // NS)],
                        outd_hbm.at[c, pl.ds(s * (HR // NS), HR // NS)])

    return body(x_halves, edges, zeros_tile, iota_hr)


def _tc_root(x, wrT, bias):
    BR = 1024
    nblk = (N + BR - 1) // BR

    def body(x_ref, wr_ref, b_ref, o_ref):
        o_ref[...] = jnp.dot(
            x_ref[...], wr_ref[...],
            preferred_element_type=jnp.float32) + b_ref[...]

    return pl.pallas_call(
        body,
        grid=(nblk,),
        in_specs=[
            pl.BlockSpec((BR, DIN), lambda i: (i, 0)),
            pl.BlockSpec((DIN, DIN), lambda i: (0, 0)),
            pl.BlockSpec((1, DIN), lambda i: (0, 0)),
        ],
        out_specs=pl.BlockSpec((BR, DIN), lambda i: (i, 0)),
        out_shape=jax.ShapeDtypeStruct((N, DIN), jnp.float32),
    )(x, wrT, bias)


def _tc_combine(agg, deg, root, wnT):
    BR = 1024
    nblk = (N + BR - 1) // BR

    def body(a_ref, d_ref, r_ref, wn_ref, o_ref):
        d = jnp.maximum(d_ref[...], 1.0)
        neigh = jnp.concatenate([a_ref[0], a_ref[1]], axis=1) / d
        o_ref[...] = jnp.dot(
            neigh, wn_ref[...],
            preferred_element_type=jnp.float32) + r_ref[...]

    return pl.pallas_call(
        body,
        grid=(nblk,),
        in_specs=[
            pl.BlockSpec((NC, BR, DH), lambda i: (0, i, 0)),
            pl.BlockSpec((BR, 1), lambda i: (i, 0)),
            pl.BlockSpec((BR, DIN), lambda i: (i, 0)),
            pl.BlockSpec((DIN, DIN), lambda i: (0, 0)),
        ],
        out_specs=pl.BlockSpec((BR, DIN), lambda i: (i, 0)),
        out_shape=jax.ShapeDtypeStruct((N, DIN), jnp.float32),
    )(agg, deg, root, wnT)


def kernel(x, edge_index, W_neigh, b_neigh, W_root, b_root):
    f32 = jnp.float32
    edges = edge_index.astype(jnp.int32)
    x = x.astype(f32)
    x_halves = x.reshape(N, NC, DH).transpose(1, 0, 2)
    zeros_tile = jnp.zeros((RPT, DH), f32)
    iota_hr = jnp.arange(HR, dtype=jnp.int32)
    agg, deg = _sc_aggregate(x_halves, edges, zeros_tile, iota_hr)
    deg = deg[0].reshape(HR * 128, 1)[:N]
    bias = (b_neigh + b_root).reshape(1, DIN).astype(f32)
    root = _tc_root(x, W_root.T.astype(f32), bias)
    return _tc_combine(agg, deg, root, W_neigh.T.astype(f32))


# speedup vs baseline: 12.2597x; 1.0428x over previous
import dataclasses
import functools

import jax
import jax.numpy as jnp
from jax import lax
from jax.experimental import pallas as pl
from jax.experimental.pallas import tpu as pltpu
from jax.experimental.pallas import tpu_sc as plsc

N = 10000
DIN = 128
E = 320000
NC = 2
NS = 16
L = 16
DH = DIN // NC
K = 80
NBUF = 5
R = N
RPT = R // NS
NCHUNK = 250
EPT = NCHUNK * K
HR = 80
NFULL = K // L

_CP = pltpu.CompilerParams(use_tc_tiling_on_sc=False)
if "needs_layout_passes" in pltpu.CompilerParams.__dataclass_fields__:
    _CP = dataclasses.replace(_CP, needs_layout_passes=False)


def _sc_aggregate(x_halves, edges, zeros_tile, iota_hr):
    mesh = plsc.VectorSubcoreMesh(core_axis_name="c", subcore_axis_name="s")

    @functools.partial(
        pl.kernel,
        mesh=mesh,
        compiler_params=_CP,
        out_type=[jax.ShapeDtypeStruct((NC, R, DH), jnp.float32),
                  jax.ShapeDtypeStruct((NC, HR, 128), jnp.float32)],
        scratch_types=[
            pltpu.VMEM((EPT,), jnp.int32),
            pltpu.VMEM((EPT,), jnp.int32),
            *[pltpu.VMEM((K, DH), jnp.float32) for _ in range(NBUF)],
            pltpu.VMEM((HR, 128), jnp.float32),
            pltpu.VMEM((HR,), jnp.int32),
            pltpu.VMEM_SHARED((R, DH), jnp.float32),
            pltpu.VMEM_SHARED((HR, 128), jnp.float32),
            *[pltpu.SemaphoreType.DMA for _ in range(2 * NBUF)],
        ],
    )
    def body(x_hbm, e_hbm, z_hbm, i_hbm, outa_hbm, outd_hbm,
             sidx, didx, *bufs):
        rows = bufs[:NBUF]
        hist = bufs[NBUF]
        iota_v = bufs[NBUF + 1]
        agg = bufs[NBUF + 2]
        sdeg = bufs[NBUF + 3]
        gsem = bufs[NBUF + 4:NBUF + 4 + NBUF]
        ssem = bufs[NBUF + 4 + NBUF:]
        c = lax.axis_index("c")
        s = lax.axis_index("s")
        xh = x_hbm.at[c]
        ones16 = jnp.ones((L,), jnp.float32)
        zeros16 = jnp.zeros((L,), jnp.float32)

        cp0 = pltpu.async_copy(e_hbm.at[0, pl.ds(s * EPT, EPT)], sidx, gsem[0])
        cp1 = pltpu.async_copy(e_hbm.at[1, pl.ds(s * EPT, EPT)], didx, gsem[1])
        cp2 = pltpu.async_copy(z_hbm, agg.at[pl.ds(s * RPT, RPT)], gsem[2])
        cp3 = pltpu.async_copy(i_hbm, iota_v, gsem[3])

        @pl.loop(0, HR)
        def _(r):
            for v in range(8):
                hist[r, pl.ds(v * L, L)] = zeros16

        cp0.wait()
        cp1.wait()
        cp2.wait()
        cp3.wait()

        @pl.when(s == 0)
        def _():
            pltpu.sync_copy(hist, sdeg)
        plsc.subcore_barrier()

        def hist_row(r):
            for v in range(NFULL):
                idx = didx[pl.ds(r * K + v * L, L)]
                plsc.addupdate_scatter(hist, [idx >> 7, idx & 127], ones16)

        for b in range(NBUF):
            pltpu.async_copy(xh.at[sidx.at[pl.ds(b * K, K)]], rows[b], gsem[b])

        @pl.loop(0, NCHUNK, step=NBUF)
        def _(j):
            for b in range(NBUF):
                pltpu.make_async_copy(xh.at[sidx.at[pl.ds(0, K)]], rows[b],
                                      gsem[b]).wait()
                pltpu.async_copy(rows[b], agg.at[didx.at[pl.ds((j + b) * K, K)]],
                                 ssem[b], add=True)
            for b in range(NBUF):
                hist_row(j + b)
            for b in range(NBUF):
                pltpu.make_async_copy(rows[b], agg.at[didx.at[pl.ds(0, K)]],
                                      ssem[b]).wait()

                @pl.when(j + NBUF + b < NCHUNK)
                def _():
                    pltpu.async_copy(
                        xh.at[sidx.at[pl.ds((j + NBUF + b) * K, K)]], rows[b],
                        gsem[b])

        plsc.subcore_barrier()
        pltpu.sync_copy(hist, sdeg.at[iota_v], add=True)
        plsc.subcore_barrier()
        pltpu.sync_copy(agg.at[pl.ds(s * RPT, RPT)],
                        outa_hbm.at[c, pl.ds(s * RPT, RPT)])
        pltpu.sync_copy(sdeg.at[pl.ds(s * (HR // NS), HR // NS)],
                        outd_hbm.at[c, pl.ds(s * (HR // NS), HR // NS)])

    return body(x_halves, edges, zeros_tile, iota_hr)


def _tc_split(x):
    BR = 2000
    nblk = N // BR

    def body(x_ref, o_ref):
        o_ref[0] = x_ref[:, :DH]
        o_ref[1] = x_ref[:, DH:]

    return pl.pallas_call(
        body,
        grid=(nblk,),
        in_specs=[pl.BlockSpec((BR, DIN), lambda i: (i, 0))],
        out_specs=pl.BlockSpec((NC, BR, DH), lambda i: (0, i, 0)),
        out_shape=jax.ShapeDtypeStruct((NC, N, DH), jnp.float32),
    )(x)


def _tc_root(x, wrT, bias):
    BR = 1024
    nblk = (N + BR - 1) // BR

    def body(x_ref, wr_ref, b_ref, o_ref):
        o_ref[...] = jnp.dot(
            x_ref[...], wr_ref[...],
            preferred_element_type=jnp.float32) + b_ref[...]

    return pl.pallas_call(
        body,
        grid=(nblk,),
        in_specs=[
            pl.BlockSpec((BR, DIN), lambda i: (i, 0)),
            pl.BlockSpec((DIN, DIN), lambda i: (0, 0)),
            pl.BlockSpec((1, DIN), lambda i: (0, 0)),
        ],
        out_specs=pl.BlockSpec((BR, DIN), lambda i: (i, 0)),
        out_shape=jax.ShapeDtypeStruct((N, DIN), jnp.float32),
    )(x, wrT, bias)


def _tc_combine(agg, deg, root, wnT):
    BR = 2048
    nblk = (N + BR - 1) // BR

    def body(a_ref, d_ref, r_ref, wn_ref, o_ref):
        d = jnp.maximum(d_ref[...], 1.0)
        neigh = jnp.concatenate([a_ref[0], a_ref[1]], axis=1) / d
        o_ref[...] = jnp.dot(
            neigh.astype(jnp.bfloat16), wn_ref[...],
            preferred_element_type=jnp.float32) + r_ref[...]

    return pl.pallas_call(
        body,
        grid=(nblk,),
        in_specs=[
            pl.BlockSpec((NC, BR, DH), lambda i: (0, i, 0)),
            pl.BlockSpec((BR, 1), lambda i: (i, 0)),
            pl.BlockSpec((BR, DIN), lambda i: (i, 0)),
            pl.BlockSpec((DIN, DIN), lambda i: (0, 0)),
        ],
        out_specs=pl.BlockSpec((BR, DIN), lambda i: (i, 0)),
        out_shape=jax.ShapeDtypeStruct((N, DIN), jnp.float32),
    )(agg, deg, root, wnT.astype(jnp.bfloat16))


def kernel(x, edge_index, W_neigh, b_neigh, W_root, b_root):
    f32 = jnp.float32
    edges = edge_index.astype(jnp.int32)
    x = x.astype(f32)
    x_halves = _tc_split(x)
    zeros_tile = jnp.zeros((RPT, DH), f32)
    iota_hr = jnp.arange(HR, dtype=jnp.int32)
    agg, deg = _sc_aggregate(x_halves, edges, zeros_tile, iota_hr)
    deg = deg[0].reshape(HR * 128, 1)[:N]
    bias = (b_neigh + b_root).reshape(1, DIN).astype(f32)
    root = _tc_root(x, W_root.T.astype(f32), bias)
    return _tc_combine(agg, deg, root, W_neigh.T.astype(f32))


# speedup vs baseline: 12.8371x; 1.0471x over previous
import dataclasses
import functools

import jax
import jax.numpy as jnp
from jax import lax
from jax.experimental import pallas as pl
from jax.experimental.pallas import tpu as pltpu
from jax.experimental.pallas import tpu_sc as plsc

N = 10000
DIN = 128
E = 320000
NC = 2
NS = 16
L = 16
DH = DIN // NC
K = 80
NBUF = 5
R = N
RPT = R // NS
NCHUNK = 250
EPT = NCHUNK * K
HR = 80
NFULL = K // L

_CP = pltpu.CompilerParams(use_tc_tiling_on_sc=False)
if "needs_layout_passes" in pltpu.CompilerParams.__dataclass_fields__:
    _CP = dataclasses.replace(_CP, needs_layout_passes=False)


def _sc_aggregate(x, edges, zeros_tile, iota_hr):
    mesh = plsc.VectorSubcoreMesh(core_axis_name="c", subcore_axis_name="s")

    @functools.partial(
        pl.kernel,
        mesh=mesh,
        compiler_params=_CP,
        out_type=[jax.ShapeDtypeStruct((NC, R, DH), jnp.float32),
                  jax.ShapeDtypeStruct((NC, HR, 128), jnp.float32),
                  jax.ShapeDtypeStruct((NC, N, DH), jnp.float32)],
        scratch_types=[
            pltpu.VMEM((EPT,), jnp.int32),
            pltpu.VMEM((EPT,), jnp.int32),
            *[pltpu.VMEM((K, DH), jnp.float32) for _ in range(NBUF)],
            pltpu.VMEM((HR, 128), jnp.float32),
            pltpu.VMEM((HR,), jnp.int32),
            pltpu.VMEM_SHARED((R, DH), jnp.float32),
            pltpu.VMEM_SHARED((HR, 128), jnp.float32),
            *[pltpu.SemaphoreType.DMA for _ in range(2 * NBUF)],
        ],
    )
    def body(x_hbm, e_hbm, z_hbm, i_hbm, outa_hbm, outd_hbm, xh_hbm,
             sidx, didx, *bufs):
        rows = bufs[:NBUF]
        hist = bufs[NBUF]
        iota_v = bufs[NBUF + 1]
        agg = bufs[NBUF + 2]
        sdeg = bufs[NBUF + 3]
        gsem = bufs[NBUF + 4:NBUF + 4 + NBUF]
        ssem = bufs[NBUF + 4 + NBUF:]
        c = lax.axis_index("c")
        s = lax.axis_index("s")
        xh = xh_hbm.at[c]
        ones16 = jnp.ones((L,), jnp.float32)
        zeros16 = jnp.zeros((L,), jnp.float32)

        cp0 = pltpu.async_copy(e_hbm.at[0, pl.ds(s * EPT, EPT)], sidx, gsem[0])
        cp1 = pltpu.async_copy(e_hbm.at[1, pl.ds(s * EPT, EPT)], didx, gsem[1])
        cp2 = pltpu.async_copy(z_hbm, agg.at[pl.ds(s * RPT, RPT)], gsem[2])
        cp3 = pltpu.async_copy(i_hbm, iota_v, gsem[3])

        @pl.loop(0, HR)
        def _(r):
            for v in range(8):
                hist[r, pl.ds(v * L, L)] = zeros16

        cp0.wait()
        cp1.wait()
        cp2.wait()
        cp3.wait()
        base = s * RPT
        for q in range(RPT // K):
            sync0 = pltpu.sync_copy
            sync0(x_hbm.at[pl.ds(base + q * K, K), pl.ds(c * DH, DH)], rows[0])
            sync0(rows[0], xh.at[pl.ds(base + q * K, K)])
        tail = RPT - (RPT // K) * K
        if tail:
            toff = base + (RPT // K) * K
            pltpu.sync_copy(
                x_hbm.at[pl.ds(toff, tail), pl.ds(c * DH, DH)],
                rows[0].at[pl.ds(0, tail)])
            pltpu.sync_copy(rows[0].at[pl.ds(0, tail)],
                            xh.at[pl.ds(toff, tail)])

        @pl.when(s == 0)
        def _():
            pltpu.sync_copy(hist, sdeg)
        plsc.subcore_barrier()

        def hist_row(r):
            for v in range(NFULL):
                idx = didx[pl.ds(r * K + v * L, L)]
                plsc.addupdate_scatter(hist, [idx >> 7, idx & 127], ones16)

        for b in range(NBUF):
            pltpu.async_copy(xh.at[sidx.at[pl.ds(b * K, K)]], rows[b], gsem[b])

        @pl.loop(0, NCHUNK, step=NBUF)
        def _(j):
            for b in range(NBUF):
                pltpu.make_async_copy(xh.at[sidx.at[pl.ds(0, K)]], rows[b],
                                      gsem[b]).wait()
                pltpu.async_copy(rows[b], agg.at[didx.at[pl.ds((j + b) * K, K)]],
                                 ssem[b], add=True)
            for b in range(NBUF):
                hist_row(j + b)
            for b in range(NBUF):
                pltpu.make_async_copy(rows[b], agg.at[didx.at[pl.ds(0, K)]],
                                      ssem[b]).wait()

                @pl.when(j + NBUF + b < NCHUNK)
                def _():
                    pltpu.async_copy(
                        xh.at[sidx.at[pl.ds((j + NBUF + b) * K, K)]], rows[b],
                        gsem[b])

        plsc.subcore_barrier()
        pltpu.sync_copy(hist, sdeg.at[iota_v], add=True)
        plsc.subcore_barrier()
        pltpu.sync_copy(agg.at[pl.ds(s * RPT, RPT)],
                        outa_hbm.at[c, pl.ds(s * RPT, RPT)])
        pltpu.sync_copy(sdeg.at[pl.ds(s * (HR // NS), HR // NS)],
                        outd_hbm.at[c, pl.ds(s * (HR // NS), HR // NS)])

    return body(x, edges, zeros_tile, iota_hr)


def _tc_root(x, wrT, bias):
    BR = 1024
    nblk = (N + BR - 1) // BR

    def body(x_ref, wr_ref, b_ref, o_ref):
        o_ref[...] = jnp.dot(
            x_ref[...], wr_ref[...],
            preferred_element_type=jnp.float32) + b_ref[...]

    return pl.pallas_call(
        body,
        grid=(nblk,),
        in_specs=[
            pl.BlockSpec((BR, DIN), lambda i: (i, 0)),
            pl.BlockSpec((DIN, DIN), lambda i: (0, 0)),
            pl.BlockSpec((1, DIN), lambda i: (0, 0)),
        ],
        out_specs=pl.BlockSpec((BR, DIN), lambda i: (i, 0)),
        out_shape=jax.ShapeDtypeStruct((N, DIN), jnp.float32),
    )(x, wrT, bias)


def _tc_combine(agg, deg, root, wnT):
    BR = 2048
    nblk = (N + BR - 1) // BR

    def body(a_ref, d_ref, r_ref, wn_ref, o_ref):
        d = jnp.maximum(d_ref[...], 1.0)
        neigh = jnp.concatenate([a_ref[0], a_ref[1]], axis=1) / d
        o_ref[...] = jnp.dot(
            neigh.astype(jnp.bfloat16), wn_ref[...],
            preferred_element_type=jnp.float32) + r_ref[...]

    return pl.pallas_call(
        body,
        grid=(nblk,),
        in_specs=[
            pl.BlockSpec((NC, BR, DH), lambda i: (0, i, 0)),
            pl.BlockSpec((BR, 1), lambda i: (i, 0)),
            pl.BlockSpec((BR, DIN), lambda i: (i, 0)),
            pl.BlockSpec((DIN, DIN), lambda i: (0, 0)),
        ],
        out_specs=pl.BlockSpec((BR, DIN), lambda i: (i, 0)),
        out_shape=jax.ShapeDtypeStruct((N, DIN), jnp.float32),
    )(agg, deg, root, wnT.astype(jnp.bfloat16))


def kernel(x, edge_index, W_neigh, b_neigh, W_root, b_root):
    f32 = jnp.float32
    edges = edge_index.astype(jnp.int32)
    x = x.astype(f32)
    zeros_tile = jnp.zeros((RPT, DH), f32)
    iota_hr = jnp.arange(HR, dtype=jnp.int32)
    agg, deg, _ = _sc_aggregate(x, edges, zeros_tile, iota_hr)
    deg = deg[0].reshape(HR * 128, 1)[:N]
    bias = (b_neigh + b_root).reshape(1, DIN).astype(f32)
    root = _tc_root(x, W_root.T.astype(f32), bias)
    return _tc_combine(agg, deg, root, W_neigh.T.astype(f32))
